# baseline (speedup 1.0000x reference)
import functools

import jax
import jax.numpy as jnp
from jax import lax
from jax.experimental import pallas as pl
from jax.experimental.pallas import tpu as pltpu
from jax.experimental.pallas import tpu_sc as plsc

NC = 2
NS = 16
BLK = 128

_HIGH = jax.lax.Precision.HIGHEST


def _mesh():
    return plsc.VectorSubcoreMesh(
        core_axis_name="c", subcore_axis_name="s", num_cores=NC, num_subcores=NS
    )


def _sc_params():
    return pltpu.CompilerParams(use_tc_tiling_on_sc=False)


def _fill_const(ref, n_rows, f, val):
    vec = jnp.full((16,), val, jnp.float32)

    def row(i, c):
        for j in range(f // 16):
            ref[i, pl.ds(j * 16, 16)] = vec
        return c

    lax.fori_loop(0, n_rows, row, 0)


def _make_deg_pass(n_blk, nacc):
    f = 16
    rpt = nacc // NS
    lag = 8

    @functools.partial(
        pl.kernel,
        out_type=jax.ShapeDtypeStruct((NC, nacc, f), jnp.float32),
        mesh=_mesh(),
        compiler_params=_sc_params(),
        scratch_types=[
            pltpu.VMEM((n_blk, BLK), jnp.int32),
            pltpu.VMEM((BLK, f), jnp.float32),
            pltpu.VMEM_SHARED((nacc, f), jnp.float32),
            pltpu.SemaphoreType.DMA,
        ],
    )
    def kern(dst_hbm, out_hbm, dst_v, ones_v, acc_sh, sem):
        cid = lax.axis_index("c")
        sid = lax.axis_index("s")
        wid = cid * NS + sid

        def drain1():
            pltpu.make_async_copy(out_hbm.at[0, pl.ds(0, BLK)], ones_v, sem).wait()

        _fill_const(ones_v, BLK, f, 0.0)
        for z in range(rpt // BLK):
            pltpu.sync_copy(ones_v, acc_sh.at[pl.ds(sid * rpt + z * BLK, BLK)])
        _fill_const(ones_v, BLK, f, 1.0)
        pltpu.sync_copy(dst_hbm.at[wid], dst_v)
        plsc.subcore_barrier()

        def body(j, c):
            pltpu.async_copy(ones_v, acc_sh.at[dst_v.at[j]], sem, add=True)

            @pl.when(j >= lag)
            def _():
                drain1()

            return c

        lax.fori_loop(0, n_blk, body, 0)
        for _ in range(lag):
            drain1()
        plsc.subcore_barrier()
        pltpu.sync_copy(
            acc_sh.at[pl.ds(sid * rpt, rpt)], out_hbm.at[cid, pl.ds(sid * rpt, rpt)]
        )

    return kern


def _make_agg_pass(n_blk, nacc, f):
    rpt = nacc // NS
    ngrp = n_blk // 4
    assert n_blk % 8 == 0 and ngrp % 2 == 0

    @functools.partial(
        pl.kernel,
        out_type=jax.ShapeDtypeStruct((NC, nacc, f), jnp.float32),
        mesh=_mesh(),
        compiler_params=_sc_params(),
        scratch_types=[
            pltpu.VMEM((n_blk, BLK), jnp.int32),
            pltpu.VMEM((n_blk, BLK), jnp.int32),
            pltpu.VMEM((4 * BLK, f), jnp.float32),
            pltpu.VMEM((4 * BLK, f), jnp.float32),
            pltpu.VMEM_SHARED((nacc, f), jnp.float32),
            pltpu.SemaphoreType.DMA,
            pltpu.SemaphoreType.DMA,
            pltpu.SemaphoreType.DMA,
            pltpu.SemaphoreType.DMA,
        ],
    )
    def kern(src_hbm, dst_hbm, table_hbm, out_hbm,
             src_v, dst_v, bank0, bank1, acc_sh, sg0, sg1, ss0, ss1):
        cid = lax.axis_index("c")
        sid = lax.axis_index("s")
        wid = cid * NS + sid

        def fire_g(g, bank, sem):
            for k in range(4):
                pltpu.async_copy(
                    table_hbm.at[src_v.at[4 * g + k]],
                    bank.at[pl.ds(k * BLK, BLK)], sem)

        def fire_s(g, bank, sem):
            for k in range(4):
                pltpu.async_copy(
                    bank.at[pl.ds(k * BLK, BLK)],
                    acc_sh.at[dst_v.at[4 * g + k]], sem, add=True)

        def drain(bank, sem):
            for k in range(4):
                pltpu.make_async_copy(
                    out_hbm.at[0, pl.ds(0, BLK)], bank.at[pl.ds(0, BLK)], sem).wait()

        _fill_const(bank0, BLK, f, 0.0)
        for z in range(rpt // BLK):
            pltpu.sync_copy(
                bank0.at[pl.ds(0, BLK)],
                acc_sh.at[pl.ds(sid * rpt + z * BLK, BLK)])
        pltpu.sync_copy(src_hbm.at[wid], src_v)
        pltpu.sync_copy(dst_hbm.at[wid], dst_v)
        plsc.subcore_barrier()

        fire_g(0, bank0, sg0)

        def body(i, c):
            g0 = 2 * i
            g1 = g0 + 1

            @pl.when(i > 0)
            def _():
                drain(bank1, ss1)

            fire_g(g1, bank1, sg1)
            drain(bank0, sg0)
            fire_s(g0, bank0, ss0)

            @pl.when(g1 + 1 < ngrp)
            def _():
                drain(bank0, ss0)
                fire_g(g1 + 1, bank0, sg0)

            drain(bank1, sg1)
            fire_s(g1, bank1, ss1)
            return c

        lax.fori_loop(0, ngrp // 2, body, 0)
        drain(bank0, ss0)
        drain(bank1, ss1)
        plsc.subcore_barrier()
        pltpu.sync_copy(
            acc_sh.at[pl.ds(sid * rpt, rpt)], out_hbm.at[cid, pl.ds(sid * rpt, rpt)]
        )

    return kern


def _dinv_of(degp_ref, n):
    deg = degp_ref[0, 0:n, :] + degp_ref[1, 0:n, :] + 1.0
    return 1.0 / jnp.sqrt(deg[:, 0:1])


def _mm1_body(n, xt_ref, w_ref, out_ref):
    out_ref[...] = jax.lax.dot_general(
        xt_ref[...], w_ref[...], (((0,), (0,)), ((), ())),
        precision=_HIGH, preferred_element_type=jnp.float32)


def _scale_body(n, xt_ref, degp_ref, out_ref):
    out_ref[...] = xt_ref[...] * _dinv_of(degp_ref, n)


def _post1_body(n, p_ref, xs_ref, degp_ref, b_ref, g_ref, be_ref, w_ref, out_ref):
    dinv = _dinv_of(degp_ref, n)
    acc = p_ref[0, 0:n, :] + p_ref[1, 0:n, :] + xs_ref[...]
    h = jnp.maximum(acc * dinv + b_ref[...], 0.0)
    m = jnp.mean(h, axis=0, keepdims=True)
    v = jnp.mean((h - m) ** 2, axis=0, keepdims=True)
    hbn = (h - m) * (1.0 / jnp.sqrt(v + 1e-5)) * g_ref[...] + be_ref[...]
    xt = jnp.dot(hbn, w_ref[...], precision=_HIGH, preferred_element_type=jnp.float32)
    out_ref[...] = xt * dinv


def _post2_body(n, p_ref, xs_ref, degp_ref, b_ref, g_ref, be_ref, w_ref,
                x1_ref, xs3_ref):
    dinv = _dinv_of(degp_ref, n)
    acc = p_ref[0, 0:n, :] + p_ref[1, 0:n, :] + xs_ref[...]
    x1f = acc * dinv + b_ref[...]
    x1 = x1f[:, 0:8]
    x1_ref[...] = x1
    x2 = jnp.maximum(x1, 0.0)
    m = jnp.mean(x2, axis=0, keepdims=True)
    v = jnp.mean((x2 - m) ** 2, axis=0, keepdims=True)
    xbn = (x2 - m) * (1.0 / jnp.sqrt(v + 1e-5)) * g_ref[...] + be_ref[...]
    xt = jnp.dot(xbn, w_ref[...], precision=_HIGH, preferred_element_type=jnp.float32)
    xs3_ref[...] = xt * dinv


def _post3_body(n, p_ref, xs_ref, degp_ref, b_ref, out_ref):
    dinv = _dinv_of(degp_ref, n)
    acc = p_ref[0, 0:n, :] + p_ref[1, 0:n, :] + xs_ref[...]
    out_ref[...] = acc * dinv + b_ref[...]


def _tc(body, out_shape, *args):
    return pl.pallas_call(
        body, out_shape=out_shape,
        compiler_params=pltpu.CompilerParams(vmem_limit_bytes=120 * 1024 * 1024),
    )(*args)


def kernel(x, edge_index, W1, b1, W2, b2, W3, b3, g1, be1, g2, be2):
    n = x.shape[-2]
    e = edge_index.shape[1]
    f1 = W1.shape[1]
    cin = x.shape[-1]
    xt2d = jnp.transpose(x.reshape(n, cin))

    t = NC * NS
    n_blk = -(-e // (t * BLK))
    n_blk = -(-n_blk // 8) * 8
    ep = t * n_blk * BLK
    nacc = -(-(n + 1) // (NS * BLK)) * (NS * BLK)

    pad = ep - e
    src_p = jnp.concatenate(
        [edge_index[0], jnp.zeros((pad,), jnp.int32)]).reshape(t, n_blk, BLK)
    dst_p = jnp.concatenate(
        [edge_index[1], jnp.full((pad,), n, jnp.int32)]).reshape(t, n_blk, BLK)

    w2p = jnp.zeros((f1, 16), jnp.float32).at[:, :8].set(W2)
    b2p = jnp.zeros((16,), jnp.float32).at[:8].set(b2)
    w3p = jnp.zeros((8, 16), jnp.float32).at[:, :3].set(W3)
    b3p = jnp.zeros((16,), jnp.float32).at[:3].set(b3)

    deg_pass = _make_deg_pass(n_blk, nacc)
    agg64 = _make_agg_pass(n_blk, nacc, f1)
    agg16 = _make_agg_pass(n_blk, nacc, 16)

    degp = deg_pass(dst_p)
    xt1 = _tc(functools.partial(_mm1_body, n),
              jax.ShapeDtypeStruct((n, f1), jnp.float32), xt2d, W1)
    xs1 = _tc(functools.partial(_scale_body, n),
              jax.ShapeDtypeStruct((n, f1), jnp.float32), xt1, degp)
    p1 = agg64(src_p, dst_p, xs1)
    xs2 = _tc(functools.partial(_post1_body, n),
              jax.ShapeDtypeStruct((n, 16), jnp.float32),
              p1, xs1, degp, b1, g1, be1, w2p)
    p2 = agg16(src_p, dst_p, xs2)
    x1, xs3 = _tc(functools.partial(_post2_body, n),
                  [jax.ShapeDtypeStruct((n, 8), jnp.float32),
                   jax.ShapeDtypeStruct((n, 16), jnp.float32)],
                  p2, xs2, degp, b2p, g2, be2, w3p)
    p3 = agg16(src_p, dst_p, xs3)
    out16 = _tc(functools.partial(_post3_body, n),
                jax.ShapeDtypeStruct((n, 16), jnp.float32),
                p3, xs3, degp, b3p)

    out = out16[:, :3].reshape(1, 1, n, 3)
    return (out, x1.reshape(1, 1, n, 8))


# speedup vs baseline: 1.8240x; 1.8240x over previous
import functools

import jax
import jax.numpy as jnp
from jax import lax
from jax.experimental import pallas as pl
from jax.experimental.pallas import tpu as pltpu
from jax.experimental.pallas import tpu_sc as plsc

NC = 2
NS = 16
BLK = 128

_HIGH = jax.lax.Precision.HIGHEST


def _mesh():
    return plsc.VectorSubcoreMesh(
        core_axis_name="c", subcore_axis_name="s", num_cores=NC, num_subcores=NS
    )


def _sc_params():
    return pltpu.CompilerParams(use_tc_tiling_on_sc=False)


def _fill_const(ref, n_rows, f, val):
    vec = jnp.full((16,), val, jnp.float32)

    def row(i, c):
        for j in range(f // 16):
            ref[i, pl.ds(j * 16, 16)] = vec
        return c

    lax.fori_loop(0, n_rows, row, 0)


def _make_deg_pass(n_blk, nacc):
    f = 16
    rpt = nacc // NS
    lag = 8

    @functools.partial(
        pl.kernel,
        out_type=jax.ShapeDtypeStruct((NC, nacc, f), jnp.float32),
        mesh=_mesh(),
        compiler_params=_sc_params(),
        scratch_types=[
            pltpu.VMEM((n_blk, BLK), jnp.int32),
            pltpu.VMEM((BLK, f), jnp.float32),
            pltpu.VMEM_SHARED((nacc, f), jnp.float32),
            pltpu.SemaphoreType.DMA,
        ],
    )
    def kern(dst_hbm, out_hbm, dst_v, ones_v, acc_sh, sem):
        cid = lax.axis_index("c")
        sid = lax.axis_index("s")
        wid = cid * NS + sid

        def drain1():
            pltpu.make_async_copy(out_hbm.at[0, pl.ds(0, BLK)], ones_v, sem).wait()

        _fill_const(ones_v, BLK, f, 0.0)
        for z in range(rpt // BLK):
            pltpu.sync_copy(ones_v, acc_sh.at[pl.ds(sid * rpt + z * BLK, BLK)])
        _fill_const(ones_v, BLK, f, 1.0)
        pltpu.sync_copy(dst_hbm.at[wid], dst_v)
        plsc.subcore_barrier()

        def body(j, c):
            pltpu.async_copy(ones_v, acc_sh.at[dst_v.at[j]], sem, add=True)

            @pl.when(j >= lag)
            def _():
                drain1()

            return c

        lax.fori_loop(0, n_blk, body, 0)
        for _ in range(lag):
            drain1()
        plsc.subcore_barrier()
        pltpu.sync_copy(
            acc_sh.at[pl.ds(sid * rpt, rpt)], out_hbm.at[cid, pl.ds(sid * rpt, rpt)]
        )

    return kern


def _make_agg_pass(n, n_blk, nacc, f, feature_split):
    rpt = nacc // NS
    tpt = n // NS
    assert n % NS == 0
    ngrp = n_blk // 4
    assert n_blk % 8 == 0 and ngrp % 2 == 0
    tshape = (NC, n, f) if feature_split else (n, f)

    @functools.partial(
        pl.kernel,
        out_type=jax.ShapeDtypeStruct((NC, nacc, f), jnp.float32),
        mesh=_mesh(),
        compiler_params=_sc_params(),
        scratch_types=[
            pltpu.VMEM((n_blk, BLK), jnp.int32),
            pltpu.VMEM((n_blk, BLK), jnp.int32),
            pltpu.VMEM((4 * BLK, f), jnp.float32),
            pltpu.VMEM((4 * BLK, f), jnp.float32),
            pltpu.VMEM_SHARED((nacc, f), jnp.float32),
            pltpu.VMEM_SHARED((n, f), jnp.float32),
            pltpu.SemaphoreType.DMA,
            pltpu.SemaphoreType.DMA,
            pltpu.SemaphoreType.DMA,
            pltpu.SemaphoreType.DMA,
        ],
    )
    def kern(src_hbm, dst_hbm, table_hbm, out_hbm,
             src_v, dst_v, bank0, bank1, acc_sh, tab_sh, sg0, sg1, ss0, ss1):
        cid = lax.axis_index("c")
        sid = lax.axis_index("s")
        wid = sid if feature_split else cid * NS + sid

        def fire_g(g, bank, sem):
            for k in range(4):
                pltpu.async_copy(
                    tab_sh.at[src_v.at[4 * g + k]],
                    bank.at[pl.ds(k * BLK, BLK)], sem)

        def fire_s(g, bank, sem):
            for k in range(4):
                pltpu.async_copy(
                    bank.at[pl.ds(k * BLK, BLK)],
                    acc_sh.at[dst_v.at[4 * g + k]], sem, add=True)

        def drain(bank, sem):
            for k in range(4):
                pltpu.make_async_copy(
                    out_hbm.at[0, pl.ds(0, BLK)], bank.at[pl.ds(0, BLK)], sem).wait()

        tab_src = (table_hbm.at[cid, pl.ds(sid * tpt, tpt)] if feature_split
                   else table_hbm.at[pl.ds(sid * tpt, tpt)])
        pltpu.async_copy(tab_src, tab_sh.at[pl.ds(sid * tpt, tpt)], sg1)
        _fill_const(bank0, BLK, f, 0.0)
        for z in range(rpt // BLK):
            pltpu.sync_copy(
                bank0.at[pl.ds(0, BLK)],
                acc_sh.at[pl.ds(sid * rpt + z * BLK, BLK)])
        pltpu.sync_copy(src_hbm.at[wid], src_v)
        pltpu.sync_copy(dst_hbm.at[wid], dst_v)
        pltpu.make_async_copy(
            tab_src, tab_sh.at[pl.ds(sid * tpt, tpt)], sg1).wait()
        plsc.subcore_barrier()

        fire_g(0, bank0, sg0)

        def body(i, c):
            g0 = 2 * i
            g1 = g0 + 1

            @pl.when(i > 0)
            def _():
                drain(bank1, ss1)

            fire_g(g1, bank1, sg1)
            drain(bank0, sg0)
            fire_s(g0, bank0, ss0)

            @pl.when(g1 + 1 < ngrp)
            def _():
                drain(bank0, ss0)
                fire_g(g1 + 1, bank0, sg0)

            drain(bank1, sg1)
            fire_s(g1, bank1, ss1)
            return c

        lax.fori_loop(0, ngrp // 2, body, 0)
        drain(bank0, ss0)
        drain(bank1, ss1)
        plsc.subcore_barrier()
        pltpu.sync_copy(
            acc_sh.at[pl.ds(sid * rpt, rpt)], out_hbm.at[cid, pl.ds(sid * rpt, rpt)]
        )

    return kern


def _dinv_of(degp_ref, n):
    deg = degp_ref[0, 0:n, :] + degp_ref[1, 0:n, :] + 1.0
    return 1.0 / jnp.sqrt(deg[:, 0:1])


def _mm1_body(n, xt_ref, w_ref, out_ref):
    out_ref[...] = jax.lax.dot_general(
        xt_ref[...], w_ref[...], (((0,), (0,)), ((), ())),
        precision=_HIGH, preferred_element_type=jnp.float32)


def _scale_body(n, fh, xt_ref, degp_ref, out_ref):
    xs = xt_ref[...] * _dinv_of(degp_ref, n)
    out_ref[0, :, :] = xs[:, 0:fh]
    out_ref[1, :, :] = xs[:, fh:2 * fh]


def _post1_body(n, p_ref, xs_ref, degp_ref, b_ref, g_ref, be_ref, w_ref, out_ref):
    dinv = _dinv_of(degp_ref, n)
    acc = jnp.concatenate(
        [p_ref[0, 0:n, :] + xs_ref[0, :, :],
         p_ref[1, 0:n, :] + xs_ref[1, :, :]], axis=1)
    h = jnp.maximum(acc * dinv + b_ref[...], 0.0)
    m = jnp.mean(h, axis=0, keepdims=True)
    v = jnp.mean((h - m) ** 2, axis=0, keepdims=True)
    hbn = (h - m) * (1.0 / jnp.sqrt(v + 1e-5)) * g_ref[...] + be_ref[...]
    xt = jnp.dot(hbn, w_ref[...], precision=_HIGH, preferred_element_type=jnp.float32)
    out_ref[...] = xt * dinv


def _post2_body(n, p_ref, xs_ref, degp_ref, b_ref, g_ref, be_ref, w_ref,
                x1_ref, xs3_ref):
    dinv = _dinv_of(degp_ref, n)
    acc = p_ref[0, 0:n, :] + p_ref[1, 0:n, :] + xs_ref[...]
    x1f = acc * dinv + b_ref[...]
    x1 = x1f[:, 0:8]
    x1_ref[...] = x1
    x2 = jnp.maximum(x1, 0.0)
    m = jnp.mean(x2, axis=0, keepdims=True)
    v = jnp.mean((x2 - m) ** 2, axis=0, keepdims=True)
    xbn = (x2 - m) * (1.0 / jnp.sqrt(v + 1e-5)) * g_ref[...] + be_ref[...]
    xt = jnp.dot(xbn, w_ref[...], precision=_HIGH, preferred_element_type=jnp.float32)
    xs3_ref[...] = xt * dinv


def _post3_body(n, p_ref, xs_ref, degp_ref, b_ref, out_ref):
    dinv = _dinv_of(degp_ref, n)
    acc = p_ref[0, 0:n, :] + p_ref[1, 0:n, :] + xs_ref[...]
    out_ref[...] = acc * dinv + b_ref[...]


def _tc(body, out_shape, *args):
    return pl.pallas_call(
        body, out_shape=out_shape,
        compiler_params=pltpu.CompilerParams(vmem_limit_bytes=120 * 1024 * 1024),
    )(*args)


def kernel(x, edge_index, W1, b1, W2, b2, W3, b3, g1, be1, g2, be2):
    n = x.shape[-2]
    e = edge_index.shape[1]
    f1 = W1.shape[1]
    cin = x.shape[-1]
    xt2d = jnp.transpose(x.reshape(n, cin))

    t = NC * NS
    n_blk = -(-e // (t * BLK))
    n_blk = -(-n_blk // 8) * 8
    ep = t * n_blk * BLK
    nacc = -(-(n + 1) // (NS * BLK)) * (NS * BLK)

    pad = ep - e
    src_flat = jnp.concatenate([edge_index[0], jnp.zeros((pad,), jnp.int32)])
    dst_flat = jnp.concatenate([edge_index[1], jnp.full((pad,), n, jnp.int32)])
    src_p = src_flat.reshape(t, n_blk, BLK)
    dst_p = dst_flat.reshape(t, n_blk, BLK)
    src_w = src_flat.reshape(NS, 2 * n_blk, BLK)
    dst_w = dst_flat.reshape(NS, 2 * n_blk, BLK)

    w2p = jnp.zeros((f1, 16), jnp.float32).at[:, :8].set(W2)
    b2p = jnp.zeros((16,), jnp.float32).at[:8].set(b2)
    w3p = jnp.zeros((8, 16), jnp.float32).at[:, :3].set(W3)
    b3p = jnp.zeros((16,), jnp.float32).at[:3].set(b3)

    fh = f1 // 2
    deg_pass = _make_deg_pass(n_blk, nacc)
    agg64 = _make_agg_pass(n, 2 * n_blk, nacc, fh, True)
    agg16 = _make_agg_pass(n, n_blk, nacc, 16, False)

    degp = deg_pass(dst_p)
    xt1 = _tc(functools.partial(_mm1_body, n),
              jax.ShapeDtypeStruct((n, f1), jnp.float32), xt2d, W1)
    xs1 = _tc(functools.partial(_scale_body, n, fh),
              jax.ShapeDtypeStruct((NC, n, fh), jnp.float32), xt1, degp)
    p1 = agg64(src_w, dst_w, xs1)
    xs2 = _tc(functools.partial(_post1_body, n),
              jax.ShapeDtypeStruct((n, 16), jnp.float32),
              p1, xs1, degp, b1, g1, be1, w2p)
    p2 = agg16(src_p, dst_p, xs2)
    x1, xs3 = _tc(functools.partial(_post2_body, n),
                  [jax.ShapeDtypeStruct((n, 8), jnp.float32),
                   jax.ShapeDtypeStruct((n, 16), jnp.float32)],
                  p2, xs2, degp, b2p, g2, be2, w3p)
    p3 = agg16(src_p, dst_p, xs3)
    out16 = _tc(functools.partial(_post3_body, n),
                jax.ShapeDtypeStruct((n, 16), jnp.float32),
                p3, xs3, degp, b3p)

    out = out16[:, :3].reshape(1, 1, n, 3)
    return (out, x1.reshape(1, 1, n, 8))


# speedup vs baseline: 2.4456x; 1.3408x over previous
import functools

import jax
import jax.numpy as jnp
from jax import lax
from jax.experimental import pallas as pl
from jax.experimental.pallas import tpu as pltpu
from jax.experimental.pallas import tpu_sc as plsc

NC = 2
NS = 16
BLK = 128

_HIGH = jax.lax.Precision.HIGHEST


def _mesh():
    return plsc.VectorSubcoreMesh(
        core_axis_name="c", subcore_axis_name="s", num_cores=NC, num_subcores=NS
    )


def _sc_params():
    return pltpu.CompilerParams(use_tc_tiling_on_sc=False)


def _fill_const(ref, n_rows, f, val):
    vec = jnp.full((16,), val, jnp.float32)

    def row(i, c):
        for j in range(f // 16):
            ref[i, pl.ds(j * 16, 16)] = vec
        return c

    lax.fori_loop(0, n_rows, row, 0)


def _make_deg_pass(n_blk, nacc):
    f = 16
    rpt = nacc // NS
    lag = 8

    @functools.partial(
        pl.kernel,
        out_type=jax.ShapeDtypeStruct((NC, nacc, f), jnp.float32),
        mesh=_mesh(),
        compiler_params=_sc_params(),
        scratch_types=[
            pltpu.VMEM((n_blk, BLK), jnp.int32),
            pltpu.VMEM((BLK, f), jnp.float32),
            pltpu.VMEM_SHARED((nacc, f), jnp.float32),
            pltpu.SemaphoreType.DMA,
        ],
    )
    def kern(dst_hbm, out_hbm, dst_v, ones_v, acc_sh, sem):
        cid = lax.axis_index("c")
        sid = lax.axis_index("s")
        wid = cid * NS + sid

        def drain1():
            pltpu.make_async_copy(out_hbm.at[0, pl.ds(0, BLK)], ones_v, sem).wait()

        _fill_const(ones_v, BLK, f, 0.0)
        for z in range(rpt // BLK):
            pltpu.sync_copy(ones_v, acc_sh.at[pl.ds(sid * rpt + z * BLK, BLK)])
        _fill_const(ones_v, BLK, f, 1.0)
        pltpu.sync_copy(dst_hbm.at[wid], dst_v)
        plsc.subcore_barrier()

        def body(j, c):
            pltpu.async_copy(ones_v, acc_sh.at[dst_v.at[j]], sem, add=True)

            @pl.when(j >= lag)
            def _():
                drain1()

            return c

        lax.fori_loop(0, n_blk, body, 0)
        for _ in range(lag):
            drain1()
        plsc.subcore_barrier()
        pltpu.sync_copy(
            acc_sh.at[pl.ds(sid * rpt, rpt)], out_hbm.at[cid, pl.ds(sid * rpt, rpt)]
        )

    return kern


def _make_agg_pass(n, n_blk, nacc, f, feature_split):
    rpt = nacc // NS
    tpt = n // NS
    assert n % NS == 0
    ngrp = n_blk // 4
    assert n_blk % 8 == 0 and ngrp % 2 == 0
    tshape = (NC, n, f) if feature_split else (n, f)

    @functools.partial(
        pl.kernel,
        out_type=jax.ShapeDtypeStruct((NC, nacc, f), jnp.float32),
        mesh=_mesh(),
        compiler_params=_sc_params(),
        scratch_types=[
            pltpu.VMEM((n_blk, BLK), jnp.int32),
            pltpu.VMEM((n_blk, BLK), jnp.int32),
            pltpu.VMEM((4 * BLK, f), jnp.float32),
            pltpu.VMEM((4 * BLK, f), jnp.float32),
            pltpu.VMEM_SHARED((nacc, f), jnp.float32),
            pltpu.VMEM_SHARED((n, f), jnp.float32),
            pltpu.SemaphoreType.DMA,
            pltpu.SemaphoreType.DMA,
            pltpu.SemaphoreType.DMA,
            pltpu.SemaphoreType.DMA,
        ],
    )
    def kern(src_hbm, dst_hbm, table_hbm, out_hbm,
             src_v, dst_v, bank0, bank1, acc_sh, tab_sh, sg0, sg1, ss0, ss1):
        cid = lax.axis_index("c")
        sid = lax.axis_index("s")
        wid = sid if feature_split else cid * NS + sid

        def fire_g(g, bank, sem):
            for k in range(4):
                pltpu.async_copy(
                    tab_sh.at[src_v.at[4 * g + k]],
                    bank.at[pl.ds(k * BLK, BLK)], sem)

        def fire_s(g, bank, sem):
            for k in range(4):
                pltpu.async_copy(
                    bank.at[pl.ds(k * BLK, BLK)],
                    acc_sh.at[dst_v.at[4 * g + k]], sem, add=True)

        def drain(bank, sem):
            for k in range(4):
                pltpu.make_async_copy(
                    out_hbm.at[0, pl.ds(0, BLK)], bank.at[pl.ds(0, BLK)], sem).wait()

        tab_src = (table_hbm.at[cid, pl.ds(sid * tpt, tpt)] if feature_split
                   else table_hbm.at[pl.ds(sid * tpt, tpt)])
        pltpu.async_copy(tab_src, tab_sh.at[pl.ds(sid * tpt, tpt)], sg1)
        pltpu.async_copy(src_hbm.at[wid], src_v, sg0)
        pltpu.async_copy(dst_hbm.at[wid], dst_v, ss1)
        _fill_const(bank0, BLK, f, 0.0)
        for z in range(rpt // BLK):
            pltpu.async_copy(
                bank0.at[pl.ds(0, BLK)],
                acc_sh.at[pl.ds(sid * rpt + z * BLK, BLK)], ss0)
        for z in range(rpt // BLK):
            pltpu.make_async_copy(
                out_hbm.at[0, pl.ds(0, BLK)], bank0.at[pl.ds(0, BLK)], ss0).wait()
        pltpu.make_async_copy(src_hbm.at[wid], src_v, sg0).wait()
        pltpu.make_async_copy(dst_hbm.at[wid], dst_v, ss1).wait()
        pltpu.make_async_copy(
            tab_src, tab_sh.at[pl.ds(sid * tpt, tpt)], sg1).wait()
        plsc.subcore_barrier()

        fire_g(0, bank0, sg0)

        def body(i, c):
            g0 = 2 * i
            g1 = g0 + 1

            @pl.when(i > 0)
            def _():
                drain(bank1, ss1)

            fire_g(g1, bank1, sg1)
            drain(bank0, sg0)
            fire_s(g0, bank0, ss0)

            @pl.when(g1 + 1 < ngrp)
            def _():
                drain(bank0, ss0)
                fire_g(g1 + 1, bank0, sg0)

            drain(bank1, sg1)
            fire_s(g1, bank1, ss1)
            return c

        lax.fori_loop(0, ngrp // 2, body, 0)
        drain(bank0, ss0)
        drain(bank1, ss1)
        plsc.subcore_barrier()
        pltpu.sync_copy(
            acc_sh.at[pl.ds(sid * rpt, rpt)], out_hbm.at[cid, pl.ds(sid * rpt, rpt)]
        )

    return kern


def _to_view(x):
    r, f = x.shape
    g = 128 // f
    x3 = jnp.reshape(x, (r // g, g, f))
    return jnp.concatenate([x3[:, j, :] for j in range(g)], axis=1)


def _from_view(v, f):
    g = 128 // f
    parts = [v[:, j * f:(j + 1) * f] for j in range(g)]
    x3 = jnp.stack(parts, axis=1)
    return jnp.reshape(x3, (v.shape[0] * g, f))


def _tile_l(vec, g):
    return jnp.concatenate([vec] * g)[None, :]


def _sel(fn, shape):
    li = lax.broadcasted_iota(jnp.int32, shape, 0)
    ci = lax.broadcasted_iota(jnp.int32, shape, 1)
    return fn(li, ci).astype(jnp.float32)


def _dot(a, b):
    return jnp.dot(a, b, precision=_HIGH, preferred_element_type=jnp.float32)


def _dinv3_of(degp_ref, n):
    degv = degp_ref[0] + degp_ref[1] + 1.0
    sel = _sel(lambda l, j: l == j * 16, (128, 8))
    deg8 = _dot(degv, sel)
    return jnp.reshape(1.0 / jnp.sqrt(deg8[0:n // 8, :]), (n // 8, 8, 1))


def _bcast_mul(x, dinv3, f):
    n = x.shape[0]
    x3 = jnp.reshape(x, (n // 8, 8, f))
    return jnp.reshape(x3 * dinv3, (n, f))


def _mm1_body(n, xt_ref, w_ref, out_ref):
    out_ref[...] = jax.lax.dot_general(
        xt_ref[...], w_ref[...], (((0,), (0,)), ((), ())),
        precision=_HIGH, preferred_element_type=jnp.float32)


def _scale_body(n, nacc, fh, xt_ref, degp_ref, out_ref):
    dinv3 = _dinv3_of(degp_ref, n)
    xs = _bcast_mul(xt_ref[...], dinv3, 2 * fh)
    out_ref[0, :, :] = _to_view(xs[:, 0:fh])
    out_ref[1, :, :] = _to_view(xs[:, fh:2 * fh])


def _post1_body(n, p_ref, xs_ref, deg16_ref, b_ref, g_ref, be_ref, w_ref,
                out_ref):
    nv = n * 32 // 128
    dv8 = (1.0 / jnp.sqrt(deg16_ref[0] + deg16_ref[1] + 1.0))[0:n * 16 // 128, :]
    d4f16 = _from_view(dv8, 64)
    dinv4 = _dot(d4f16, _sel(lambda l, l2: l == 16 * (l2 // 32), (64, 128)))
    hs, ms, rs = [], [], []
    for c in range(NC):
        accv = p_ref[c, 0:nv, :] + xs_ref[c, :, :]
        bv = _tile_l(b_ref[pl.ds(c * 32, 32)], 4)
        h = jnp.maximum(accv * dinv4 + bv, 0.0)
        hs.append(h)
    fold = _sel(lambda l, c: (l % 32) == c, (128, 32)) / 4.0
    for c in range(NC):
        m32 = _dot(jnp.mean(hs[c], axis=0, keepdims=True), fold)
        mv = _dot(m32, _sel(lambda c2, l: (l % 32) == c2, (32, 128)))
        d = hs[c] - mv
        v32 = _dot(jnp.mean(d * d, axis=0, keepdims=True), fold)
        rstd = 1.0 / jnp.sqrt(v32 + 1e-5)
        gv = _dot(rstd * g_ref[pl.ds(c * 32, 32)][None, :],
                  _sel(lambda c2, l: (l % 32) == c2, (32, 128)))
        bev = _tile_l(be_ref[pl.ds(c * 32, 32)], 4)
        hbn = d * gv + bev
        wc = w_ref[pl.ds(c * 32, 32), :]
        bd = (jnp.concatenate([jnp.concatenate([wc] * 4, axis=1)] * 4, axis=0)
              * _sel(lambda l, l2: (l // 32) == (l2 // 16), (128, 64)))
        ms.append(_dot(hbn, bd))
    xt = ms[0] + ms[1]
    dindot = _sel(lambda l, l2: l == 32 * (l2 // 16), (128, 64))
    xsv = xt * _dot(dinv4, dindot)
    x3 = jnp.reshape(xsv, (nv // 2, 2, 64))
    out_ref[...] = jnp.concatenate([x3[:, 0, :], x3[:, 1, :]], axis=1)


def _post2_body(n, p_ref, xs_ref, deg16_ref, b_ref, g_ref, be_ref, w_ref,
                x1_ref, xs3_ref):
    nv = n * 16 // 128
    dinv8 = (1.0 / jnp.sqrt(deg16_ref[0] + deg16_ref[1] + 1.0))[0:nv, :]
    accv = p_ref[0, 0:nv, :] + p_ref[1, 0:nv, :] + xs_ref[...]
    x1v = accv * dinv8 + _tile_l(b_ref[...], 8)
    x1_ref[...] = x1v
    x2 = jnp.maximum(x1v, 0.0)
    fold = _sel(lambda l, c: (l % 16) == c, (128, 16)) / 8.0
    unfold = _sel(lambda c, l: (l % 16) == c, (16, 128))
    m16 = _dot(jnp.mean(x2, axis=0, keepdims=True), fold)
    mv = _dot(m16, unfold)
    d = x2 - mv
    v16 = _dot(jnp.mean(d * d, axis=0, keepdims=True), fold)
    gv = _dot((1.0 / jnp.sqrt(v16 + 1e-5)) * g_ref[...][None, :], unfold)
    xbn = d * gv + _tile_l(be_ref[...], 8)
    bd = (jnp.concatenate([jnp.concatenate([w_ref[...]] * 8, axis=1)] * 8, axis=0)
          * _sel(lambda l, l2: (l // 16) == (l2 // 16), (128, 128)))
    xs3_ref[...] = _dot(xbn, bd) * dinv8


def _post3_body(n, p_ref, xs_ref, deg16_ref, b_ref, out_ref):
    nv = n * 16 // 128
    dinv8 = (1.0 / jnp.sqrt(deg16_ref[0] + deg16_ref[1] + 1.0))[0:nv, :]
    accv = p_ref[0, 0:nv, :] + p_ref[1, 0:nv, :] + xs_ref[...]
    out_ref[...] = accv * dinv8 + _tile_l(b_ref[...], 8)


def _tc(body, out_shape, *args):
    return pl.pallas_call(
        body, out_shape=out_shape,
        compiler_params=pltpu.CompilerParams(vmem_limit_bytes=120 * 1024 * 1024),
    )(*args)


def kernel(x, edge_index, W1, b1, W2, b2, W3, b3, g1, be1, g2, be2):
    n = x.shape[-2]
    e = edge_index.shape[1]
    f1 = W1.shape[1]
    cin = x.shape[-1]
    xt2d = jnp.transpose(x.reshape(n, cin))

    t = NC * NS
    n_blk = -(-e // (t * BLK))
    n_blk = -(-n_blk // 8) * 8
    ep = t * n_blk * BLK
    nacc = -(-(n + 1) // (NS * BLK)) * (NS * BLK)

    pad = ep - e
    src_flat = jnp.concatenate([edge_index[0], jnp.zeros((pad,), jnp.int32)])
    dst_flat = jnp.concatenate([edge_index[1], jnp.full((pad,), n, jnp.int32)])
    src_p = src_flat.reshape(t, n_blk, BLK)
    dst_p = dst_flat.reshape(t, n_blk, BLK)
    src_w = src_flat.reshape(NS, 2 * n_blk, BLK)
    dst_w = dst_flat.reshape(NS, 2 * n_blk, BLK)

    w2p = jnp.zeros((f1, 16), jnp.float32).at[:, :8].set(W2)
    b2p = jnp.zeros((16,), jnp.float32).at[:8].set(b2)
    w3pp = jnp.zeros((16, 16), jnp.float32).at[:8, :3].set(W3)
    b3p = jnp.zeros((16,), jnp.float32).at[:3].set(b3)
    g2p = jnp.zeros((16,), jnp.float32).at[:8].set(g2)
    be2p = jnp.zeros((16,), jnp.float32).at[:8].set(be2)

    fh = f1 // 2
    deg_pass = _make_deg_pass(n_blk, nacc)
    agg64 = _make_agg_pass(n, 2 * n_blk, nacc, fh, True)
    agg16 = _make_agg_pass(n, n_blk, nacc, 16, False)

    r16 = n * 16 // 128
    ra16 = nacc * 16 // 128
    ra32 = nacc * 32 // 128

    deg16 = deg_pass(dst_p).reshape(NC, ra16, 128)
    xt1 = _tc(functools.partial(_mm1_body, n),
              jax.ShapeDtypeStruct((n, f1), jnp.float32), xt2d, W1)
    xs1 = _tc(functools.partial(_scale_body, n, nacc, fh),
              jax.ShapeDtypeStruct((NC, n * fh // 128, 128), jnp.float32),
              xt1, deg16)
    p1 = agg64(src_w, dst_w, xs1.reshape(NC, n, fh))
    xs2 = _tc(functools.partial(_post1_body, n),
              jax.ShapeDtypeStruct((r16, 128), jnp.float32),
              p1.reshape(NC, ra32, 128), xs1, deg16, b1, g1, be1, w2p)
    p2 = agg16(src_p, dst_p, xs2.reshape(n, 16))
    x1v, xs3 = _tc(functools.partial(_post2_body, n),
                   [jax.ShapeDtypeStruct((r16, 128), jnp.float32),
                    jax.ShapeDtypeStruct((r16, 128), jnp.float32)],
                   p2.reshape(NC, ra16, 128), xs2, deg16, b2p, g2p, be2p, w3pp)
    p3 = agg16(src_p, dst_p, xs3.reshape(n, 16))
    out16 = _tc(functools.partial(_post3_body, n),
                jax.ShapeDtypeStruct((r16, 128), jnp.float32),
                p3.reshape(NC, ra16, 128), xs3, deg16, b3p)

    out = out16.reshape(n, 16)[:, :3].reshape(1, 1, n, 3)
    x1 = x1v.reshape(n, 16)[:, :8].reshape(1, 1, n, 8)
    return (out, x1)


# speedup vs baseline: 2.4465x; 1.0004x over previous
import functools

import jax
import jax.numpy as jnp
from jax import lax
from jax.experimental import pallas as pl
from jax.experimental.pallas import tpu as pltpu
from jax.experimental.pallas import tpu_sc as plsc

NC = 2
NS = 16
BLK = 128

_HIGH = jax.lax.Precision.HIGHEST


def _mesh():
    return plsc.VectorSubcoreMesh(
        core_axis_name="c", subcore_axis_name="s", num_cores=NC, num_subcores=NS
    )


def _sc_params():
    return pltpu.CompilerParams(use_tc_tiling_on_sc=False)


def _fill_const(ref, n_rows, f, val):
    vec = jnp.full((16,), val, jnp.float32)

    def row(i, c):
        for j in range(f // 16):
            ref[i, pl.ds(j * 16, 16)] = vec
        return c

    lax.fori_loop(0, n_rows, row, 0)


def _make_deg_pass(n_blk, nacc):
    f = 16
    rpt = nacc // NS
    lag = 8

    @functools.partial(
        pl.kernel,
        out_type=jax.ShapeDtypeStruct((NC, nacc, f), jnp.float32),
        mesh=_mesh(),
        compiler_params=_sc_params(),
        scratch_types=[
            pltpu.VMEM((n_blk, BLK), jnp.int32),
            pltpu.VMEM((BLK, f), jnp.float32),
            pltpu.VMEM_SHARED((nacc, f), jnp.float32),
            pltpu.SemaphoreType.DMA,
        ],
    )
    def kern(dst_hbm, out_hbm, dst_v, ones_v, acc_sh, sem):
        cid = lax.axis_index("c")
        sid = lax.axis_index("s")
        wid = cid * NS + sid

        def drain1():
            pltpu.make_async_copy(out_hbm.at[0, pl.ds(0, BLK)], ones_v, sem).wait()

        _fill_const(ones_v, BLK, f, 0.0)
        for z in range(rpt // BLK):
            pltpu.sync_copy(ones_v, acc_sh.at[pl.ds(sid * rpt + z * BLK, BLK)])
        _fill_const(ones_v, BLK, f, 1.0)
        pltpu.sync_copy(dst_hbm.at[wid], dst_v)
        plsc.subcore_barrier()

        def body(j, c):
            pltpu.async_copy(ones_v, acc_sh.at[dst_v.at[j]], sem, add=True)

            @pl.when(j >= lag)
            def _():
                drain1()

            return c

        lax.fori_loop(0, n_blk, body, 0)
        for _ in range(lag):
            drain1()
        plsc.subcore_barrier()
        pltpu.sync_copy(
            acc_sh.at[pl.ds(sid * rpt, rpt)], out_hbm.at[cid, pl.ds(sid * rpt, rpt)]
        )

    return kern


def _make_agg_pass(n, n_blk, nacc, f, feature_split):
    rpt = nacc // NS
    tpt = n // NS
    assert n % NS == 0
    ngrp = n_blk // 4
    assert n_blk % 8 == 0 and ngrp % 2 == 0

    @functools.partial(
        pl.kernel,
        out_type=jax.ShapeDtypeStruct((NC, nacc, f), jnp.float32),
        mesh=_mesh(),
        compiler_params=_sc_params(),
        scratch_types=[
            pltpu.VMEM((n_blk, BLK), jnp.int32),
            pltpu.VMEM((n_blk, BLK), jnp.int32),
            pltpu.VMEM((4 * BLK, f), jnp.float32),
            pltpu.VMEM((4 * BLK, f), jnp.float32),
            pltpu.VMEM_SHARED((nacc, f), jnp.float32),
            pltpu.VMEM_SHARED((n, f), jnp.float32),
            pltpu.SemaphoreType.DMA,
            pltpu.SemaphoreType.DMA,
            pltpu.SemaphoreType.DMA,
            pltpu.SemaphoreType.DMA,
        ],
    )
    def kern(src_hbm, dst_hbm, table_hbm, out_hbm,
             src_v, dst_v, bank0, bank1, acc_sh, tab_sh, sg0, sg1, ss0, ss1):
        cid = lax.axis_index("c")
        sid = lax.axis_index("s")
        wid = sid if feature_split else cid * NS + sid

        def fire_g(g, bank, sem):
            for k in range(4):
                pltpu.async_copy(
                    tab_sh.at[src_v.at[4 * g + k]],
                    bank.at[pl.ds(k * BLK, BLK)], sem)

        def fire_s(g, bank, sem):
            for k in range(4):
                pltpu.async_copy(
                    bank.at[pl.ds(k * BLK, BLK)],
                    acc_sh.at[dst_v.at[4 * g + k]], sem, add=True)

        def drain(bank, sem):
            for k in range(4):
                pltpu.make_async_copy(
                    out_hbm.at[0, pl.ds(0, BLK)], bank.at[pl.ds(0, BLK)], sem).wait()

        tab_src = (table_hbm.at[cid, pl.ds(sid * tpt, tpt)] if feature_split
                   else table_hbm.at[pl.ds(sid * tpt, tpt)])
        pltpu.async_copy(tab_src, tab_sh.at[pl.ds(sid * tpt, tpt)], sg1)
        pltpu.async_copy(src_hbm.at[wid], src_v, sg0)
        pltpu.async_copy(dst_hbm.at[wid], dst_v, ss1)
        _fill_const(bank0, BLK, f, 0.0)
        for z in range(rpt // BLK):
            pltpu.async_copy(
                bank0.at[pl.ds(0, BLK)],
                acc_sh.at[pl.ds(sid * rpt + z * BLK, BLK)], ss0)
        for z in range(rpt // BLK):
            pltpu.make_async_copy(
                out_hbm.at[0, pl.ds(0, BLK)], bank0.at[pl.ds(0, BLK)], ss0).wait()
        pltpu.make_async_copy(src_hbm.at[wid], src_v, sg0).wait()
        pltpu.make_async_copy(dst_hbm.at[wid], dst_v, ss1).wait()
        pltpu.make_async_copy(
            tab_src, tab_sh.at[pl.ds(sid * tpt, tpt)], sg1).wait()
        plsc.subcore_barrier()

        fire_g(0, bank0, sg0)

        def body(i, c):
            g0 = 2 * i
            g1 = g0 + 1

            @pl.when(i > 0)
            def _():
                drain(bank1, ss1)

            fire_g(g1, bank1, sg1)
            drain(bank0, sg0)
            fire_s(g0, bank0, ss0)

            @pl.when(g1 + 1 < ngrp)
            def _():
                drain(bank0, ss0)
                fire_g(g1 + 1, bank0, sg0)

            drain(bank1, sg1)
            fire_s(g1, bank1, ss1)
            return c

        lax.fori_loop(0, ngrp // 2, body, 0)
        drain(bank0, ss0)
        drain(bank1, ss1)
        plsc.subcore_barrier()
        pltpu.sync_copy(
            acc_sh.at[pl.ds(sid * rpt, rpt)], out_hbm.at[cid, pl.ds(sid * rpt, rpt)]
        )

    return kern


def _to_view(x):
    r, f = x.shape
    g = 128 // f
    x3 = jnp.reshape(x, (r // g, g, f))
    return jnp.concatenate([x3[:, j, :] for j in range(g)], axis=1)


def _from_view(v, f):
    g = 128 // f
    parts = [v[:, j * f:(j + 1) * f] for j in range(g)]
    x3 = jnp.stack(parts, axis=1)
    return jnp.reshape(x3, (v.shape[0] * g, f))


def _tile_l(vec, g):
    return jnp.concatenate([vec] * g)[None, :]


def _sel(fn, shape):
    li = lax.broadcasted_iota(jnp.int32, shape, 0)
    ci = lax.broadcasted_iota(jnp.int32, shape, 1)
    return fn(li, ci).astype(jnp.float32)


def _dot(a, b):
    return jnp.dot(a, b, precision=_HIGH, preferred_element_type=jnp.float32)


def _dinv3_of(degp_ref, n):
    degv = degp_ref[0] + degp_ref[1] + 1.0
    sel = _sel(lambda l, j: l == j * 16, (128, 8))
    deg8 = _dot(degv, sel)
    return jnp.reshape(1.0 / jnp.sqrt(deg8[0:n // 8, :]), (n // 8, 8, 1))


def _bcast_mul(x, dinv3, f):
    n = x.shape[0]
    x3 = jnp.reshape(x, (n // 8, 8, f))
    return jnp.reshape(x3 * dinv3, (n, f))


def _mm1_body(n, xt_ref, w_ref, out_ref):
    out_ref[...] = jax.lax.dot_general(
        xt_ref[...], w_ref[...], (((0,), (0,)), ((), ())),
        precision=_HIGH, preferred_element_type=jnp.float32)


def _scale_body(n, nacc, fh, xt_ref, degp_ref, out_ref):
    dinv3 = _dinv3_of(degp_ref, n)
    xs = _bcast_mul(xt_ref[...], dinv3, 2 * fh)
    out_ref[0, :, :] = _to_view(xs[:, 0:fh])
    out_ref[1, :, :] = _to_view(xs[:, fh:2 * fh])


def _post1_body(n, p_ref, xs_ref, deg16_ref, b_ref, g_ref, be_ref, w_ref,
                out_ref):
    nv = n * 32 // 128
    dv8 = (1.0 / jnp.sqrt(deg16_ref[0] + deg16_ref[1] + 1.0))[0:n * 16 // 128, :]
    d4f16 = _from_view(dv8, 64)
    dinv4 = _dot(d4f16, _sel(lambda l, l2: l == 16 * (l2 // 32), (64, 128)))
    hs, ms = [], []
    for c in range(NC):
        accv = p_ref[c, 0:nv, :] + xs_ref[c, :, :]
        bv = _tile_l(b_ref[pl.ds(c * 32, 32)], 4)
        h = jnp.maximum(accv * dinv4 + bv, 0.0)
        hs.append(h)
    fold = _sel(lambda l, c: (l % 32) == c, (128, 32)) / 4.0
    for c in range(NC):
        m32 = _dot(jnp.mean(hs[c], axis=0, keepdims=True), fold)
        mv = _dot(m32, _sel(lambda c2, l: (l % 32) == c2, (32, 128)))
        d = hs[c] - mv
        v32 = _dot(jnp.mean(d * d, axis=0, keepdims=True), fold)
        rstd = 1.0 / jnp.sqrt(v32 + 1e-5)
        gv = _dot(rstd * g_ref[pl.ds(c * 32, 32)][None, :],
                  _sel(lambda c2, l: (l % 32) == c2, (32, 128)))
        bev = _tile_l(be_ref[pl.ds(c * 32, 32)], 4)
        hbn = d * gv + bev
        wc = w_ref[pl.ds(c * 32, 32), :]
        bd = (jnp.concatenate([jnp.concatenate([wc] * 4, axis=1)] * 4, axis=0)
              * _sel(lambda l, l2: (l // 32) == (l2 // 16), (128, 64)))
        ms.append(_dot(hbn, bd))
    xt = ms[0] + ms[1]
    dindot = _sel(lambda l, l2: l == 32 * (l2 // 16), (128, 64))
    xsv = xt * _dot(dinv4, dindot)
    x3 = jnp.reshape(xsv, (nv // 2, 2, 64))
    out_ref[...] = jnp.concatenate([x3[:, 0, :], x3[:, 1, :]], axis=1)


def _post2_body(n, p_ref, xs_ref, deg16_ref, b_ref, g_ref, be_ref, w_ref,
                x1_ref, xs3_ref):
    nv = n * 16 // 128
    dinv8 = (1.0 / jnp.sqrt(deg16_ref[0] + deg16_ref[1] + 1.0))[0:nv, :]
    accv = p_ref[0, 0:nv, :] + p_ref[1, 0:nv, :] + xs_ref[...]
    x1v = accv * dinv8 + _tile_l(b_ref[...], 8)
    x1_ref[...] = x1v
    x2 = jnp.maximum(x1v, 0.0)
    fold = _sel(lambda l, c: (l % 16) == c, (128, 16)) / 8.0
    unfold = _sel(lambda c, l: (l % 16) == c, (16, 128))
    m16 = _dot(jnp.mean(x2, axis=0, keepdims=True), fold)
    mv = _dot(m16, unfold)
    d = x2 - mv
    v16 = _dot(jnp.mean(d * d, axis=0, keepdims=True), fold)
    gv = _dot((1.0 / jnp.sqrt(v16 + 1e-5)) * g_ref[...][None, :], unfold)
    xbn = d * gv + _tile_l(be_ref[...], 8)
    bd = (jnp.concatenate([jnp.concatenate([w_ref[...]] * 8, axis=1)] * 8, axis=0)
          * _sel(lambda l, l2: (l // 16) == (l2 // 16), (128, 128)))
    xs3_ref[...] = _dot(xbn, bd) * dinv8


def _post3_body(n, p_ref, xs_ref, deg16_ref, b_ref, out_ref):
    nv = n * 16 // 128
    dinv8 = (1.0 / jnp.sqrt(deg16_ref[0] + deg16_ref[1] + 1.0))[0:nv, :]
    accv = p_ref[0, 0:nv, :] + p_ref[1, 0:nv, :] + xs_ref[...]
    out_ref[...] = accv * dinv8 + _tile_l(b_ref[...], 8)


def _tc(body, out_shape, *args):
    return pl.pallas_call(
        body, out_shape=out_shape,
        compiler_params=pltpu.CompilerParams(vmem_limit_bytes=62 * 1024 * 1024),
    )(*args)


def kernel(x, edge_index, W1, b1, W2, b2, W3, b3, g1, be1, g2, be2):
    n = x.shape[-2]
    e = edge_index.shape[1]
    f1 = W1.shape[1]
    cin = x.shape[-1]
    xt2d = jnp.transpose(x.reshape(n, cin))

    t = NC * NS
    n_blk = -(-e // (t * BLK))
    n_blk = -(-n_blk // 8) * 8
    ep = t * n_blk * BLK
    nacc = -(-(n + 1) // (NS * BLK)) * (NS * BLK)

    pad = ep - e
    src_flat = jnp.concatenate([edge_index[0], jnp.zeros((pad,), jnp.int32)])
    dst_flat = jnp.concatenate([edge_index[1], jnp.full((pad,), n, jnp.int32)])
    src_p = src_flat.reshape(t, n_blk, BLK)
    dst_p = dst_flat.reshape(t, n_blk, BLK)
    src_w = src_flat.reshape(NS, 2 * n_blk, BLK)
    dst_w = dst_flat.reshape(NS, 2 * n_blk, BLK)

    w2p = jnp.zeros((f1, 16), jnp.float32).at[:, :8].set(W2)
    b2p = jnp.zeros((16,), jnp.float32).at[:8].set(b2)
    w3pp = jnp.zeros((16, 16), jnp.float32).at[:8, :3].set(W3)
    b3p = jnp.zeros((16,), jnp.float32).at[:3].set(b3)
    g2p = jnp.zeros((16,), jnp.float32).at[:8].set(g2)
    be2p = jnp.zeros((16,), jnp.float32).at[:8].set(be2)

    fh = f1 // 2
    deg_pass = _make_deg_pass(n_blk, nacc)
    agg64 = _make_agg_pass(n, 2 * n_blk, nacc, fh, True)
    agg16 = _make_agg_pass(n, n_blk, nacc, 16, False)

    r16 = n * 16 // 128
    ra16 = nacc * 16 // 128
    ra32 = nacc * 32 // 128

    deg16 = deg_pass(dst_p).reshape(NC, ra16, 128)
    xt1 = _tc(functools.partial(_mm1_body, n),
              jax.ShapeDtypeStruct((n, f1), jnp.float32), xt2d, W1)
    xs1 = _tc(functools.partial(_scale_body, n, nacc, fh),
              jax.ShapeDtypeStruct((NC, n * fh // 128, 128), jnp.float32),
              xt1, deg16)
    p1 = agg64(src_w, dst_w, xs1.reshape(NC, n, fh))
    xs2 = _tc(functools.partial(_post1_body, n),
              jax.ShapeDtypeStruct((r16, 128), jnp.float32),
              p1.reshape(NC, ra32, 128), xs1, deg16, b1, g1, be1, w2p)
    p2 = agg16(src_p, dst_p, xs2.reshape(n, 16))
    x1v, xs3 = _tc(functools.partial(_post2_body, n),
                   [jax.ShapeDtypeStruct((r16, 128), jnp.float32),
                    jax.ShapeDtypeStruct((r16, 128), jnp.float32)],
                   p2.reshape(NC, ra16, 128), xs2, deg16, b2p, g2p, be2p, w3pp)
    p3 = agg16(src_p, dst_p, xs3.reshape(n, 16))
    out16 = _tc(functools.partial(_post3_body, n),
                jax.ShapeDtypeStruct((r16, 128), jnp.float32),
                p3.reshape(NC, ra16, 128), xs3, deg16, b3p)

    out = out16.reshape(n, 16)[:, :3].reshape(1, 1, n, 3)
    x1 = x1v.reshape(n, 16)[:, :8].reshape(1, 1, n, 8)
    return (out, x1)
